# Initial kernel scaffold; baseline (speedup 1.0000x reference)
import jax
import jax.numpy as jnp
from jax.experimental import pallas as pl


def kernel(feat, edge_index, efeat):
    raise NotImplementedError("write your pallas kernel here")


# speedup vs baseline: 2.4263x; 2.4263x over previous
import functools

import jax
import jax.numpy as jnp
from jax import lax
from jax.experimental import pallas as pl
from jax.experimental.pallas import tpu as pltpu
from jax.experimental.pallas import tpu_sc as plsc

N_NODES = 10000
D_FEAT = 256
DH = 128
N_EDGES = 160000
NSUB = 16
B = 80
EPW = N_EDGES // NSUB
CHUNKS = EPW // B
RPW = 624
TAIL = N_NODES - RPW * NSUB
TAIL0 = RPW * NSUB
LANES = 16


def _gine_sc(feat_a, feat_b, src, dst, efeat, out,
             sidx, didx, gath, ebuf, acc, sem):
    cid = lax.axis_index("c")
    sid = lax.axis_index("s")

    r0 = sid * RPW

    @pl.when(cid == 0)
    def _():
        pltpu.sync_copy(feat_a.at[pl.ds(r0, RPW)], acc.at[pl.ds(r0, RPW)])

    @pl.when(cid == 1)
    def _():
        pltpu.sync_copy(feat_b.at[pl.ds(r0, RPW)], acc.at[pl.ds(r0, RPW)])

    @pl.when(jnp.logical_and(cid == 0, sid == NSUB - 1))
    def _():
        pltpu.sync_copy(feat_a.at[pl.ds(TAIL0, TAIL)],
                        acc.at[pl.ds(TAIL0, TAIL)])

    @pl.when(jnp.logical_and(cid == 1, sid == NSUB - 1))
    def _():
        pltpu.sync_copy(feat_b.at[pl.ds(TAIL0, TAIL)],
                        acc.at[pl.ds(TAIL0, TAIL)])

    plsc.subcore_barrier()

    e0 = sid * EPW

    def body(g, carry):
        base = pl.multiple_of(e0 + g * B, B)
        pltpu.sync_copy(src.at[pl.ds(base, B)], sidx)
        pltpu.sync_copy(dst.at[pl.ds(base, B)], didx)

        @pl.when(cid == 0)
        def _():
            pltpu.async_copy(feat_a.at[sidx], gath, sem).wait()
            pltpu.sync_copy(efeat.at[pl.ds(base, B), pl.ds(0, DH)], ebuf)

        @pl.when(cid == 1)
        def _():
            pltpu.async_copy(feat_b.at[sidx], gath, sem).wait()
            pltpu.sync_copy(efeat.at[pl.ds(base, B), pl.ds(DH, DH)], ebuf)

        def crow(r, c2):
            for j in range(DH // LANES):
                s_ = pl.ds(j * LANES, LANES)
                gath[r, s_] = jnp.maximum(gath[r, s_] + ebuf[r, s_], 0.0)
            return c2

        lax.fori_loop(0, B, crow, 0, unroll=False)

        pltpu.sync_copy(gath, acc.at[didx], add=True)
        return carry

    lax.fori_loop(0, CHUNKS, body, 0, unroll=False)

    plsc.subcore_barrier()

    @pl.when(cid == 0)
    def _():
        pltpu.sync_copy(acc.at[pl.ds(r0, RPW)],
                        out.at[pl.ds(r0, RPW), pl.ds(0, DH)])

    @pl.when(cid == 1)
    def _():
        pltpu.sync_copy(acc.at[pl.ds(r0, RPW)],
                        out.at[pl.ds(r0, RPW), pl.ds(DH, DH)])

    @pl.when(jnp.logical_and(cid == 0, sid == NSUB - 1))
    def _():
        pltpu.sync_copy(acc.at[pl.ds(TAIL0, TAIL)],
                        out.at[pl.ds(TAIL0, TAIL), pl.ds(0, DH)])

    @pl.when(jnp.logical_and(cid == 1, sid == NSUB - 1))
    def _():
        pltpu.sync_copy(acc.at[pl.ds(TAIL0, TAIL)],
                        out.at[pl.ds(TAIL0, TAIL), pl.ds(DH, DH)])


_mesh = plsc.VectorSubcoreMesh(core_axis_name="c", subcore_axis_name="s")

_gine_call = functools.partial(
    pl.kernel,
    out_type=jax.ShapeDtypeStruct((N_NODES, D_FEAT), jnp.float32),
    mesh=_mesh,
    scratch_types=[
        pltpu.VMEM((B,), jnp.int32),
        pltpu.VMEM((B,), jnp.int32),
        pltpu.VMEM((B, DH), jnp.float32),
        pltpu.VMEM((B, DH), jnp.float32),
        pltpu.VMEM_SHARED((N_NODES, DH), jnp.float32),
        pltpu.SemaphoreType.DMA,
    ],
)(_gine_sc)


@jax.jit
def kernel(feat, edge_index, efeat):
    src = edge_index[0].astype(jnp.int32)
    dst = edge_index[1].astype(jnp.int32)
    feat_a = feat[:, :DH]
    feat_b = feat[:, DH:]
    return _gine_call(feat_a, feat_b, src, dst, efeat)


# speedup vs baseline: 3.0007x; 1.2367x over previous
import functools

import jax
import jax.numpy as jnp
from jax import lax
from jax.experimental import pallas as pl
from jax.experimental.pallas import tpu as pltpu
from jax.experimental.pallas import tpu_sc as plsc

N_NODES = 10000
D_FEAT = 256
DH = 128
N_EDGES = 160000
NSUB = 16
B = 80
EPW = N_EDGES // NSUB
CHUNKS = EPW // B
RPW = 624
TAIL = N_NODES - RPW * NSUB
TAIL0 = RPW * NSUB
LANES = 16


def _gine_sc(feat_s, src, dst, efeat3, out3,
             sidx, didx, gath, ebuf, acc,
             sem_g0, sem_g1, sem_e0, sem_e1, sem_d0, sem_d1):
    cid = lax.axis_index("c")
    sid = lax.axis_index("s")
    e0 = sid * EPW

    sem_g = (sem_g0, sem_g1)
    sem_e = (sem_e0, sem_e1)
    sem_d = (sem_d0, sem_d1)

    def start(g, b):
        base = pl.multiple_of(e0 + g * B, B)
        pltpu.sync_copy(src.at[pl.ds(base, B)], sidx[b])
        pltpu.async_copy(dst.at[pl.ds(base, B)], didx[b], sem_d[b])
        pltpu.async_copy(feat_s.at[cid].at[sidx[b]], gath[b], sem_g[b])
        pltpu.async_copy(efeat3.at[pl.ds(base, B), cid], ebuf[b], sem_e[b])

    def finish(g, b):
        pltpu.make_async_copy(feat_s.at[cid].at[didx[b]], gath[b],
                              sem_g[b]).wait()
        pltpu.make_async_copy(efeat3.at[pl.ds(0, B), cid], ebuf[b],
                              sem_e[b]).wait()

        def crow(r, c2):
            for j in range(DH // LANES):
                s_ = pl.ds(j * LANES, LANES)
                gath[b][r, s_] = jnp.maximum(gath[b][r, s_] + ebuf[b][r, s_],
                                             0.0)
            return c2

        lax.fori_loop(0, B, crow, 0, unroll=False)

        pltpu.make_async_copy(dst.at[pl.ds(0, B)], didx[b], sem_d[b]).wait()
        pltpu.sync_copy(gath[b], acc.at[didx[b]], add=True)

    start(0, 0)

    r0 = sid * RPW
    pltpu.sync_copy(feat_s.at[cid].at[pl.ds(r0, RPW)], acc.at[pl.ds(r0, RPW)])

    @pl.when(sid == NSUB - 1)
    def _():
        pltpu.sync_copy(feat_s.at[cid].at[pl.ds(TAIL0, TAIL)],
                        acc.at[pl.ds(TAIL0, TAIL)])

    plsc.subcore_barrier()

    def pair(k, carry):
        g = k * 2
        start(g + 1, 1)
        finish(g, 0)
        start(g + 2, 0)
        finish(g + 1, 1)
        return carry

    lax.fori_loop(0, (CHUNKS - 1) // 2, pair, 0, unroll=False)
    finish(CHUNKS - 1, 0)

    plsc.subcore_barrier()

    pltpu.sync_copy(acc.at[pl.ds(r0, RPW)], out3.at[pl.ds(r0, RPW), cid])

    @pl.when(sid == NSUB - 1)
    def _():
        pltpu.sync_copy(acc.at[pl.ds(TAIL0, TAIL)],
                        out3.at[pl.ds(TAIL0, TAIL), cid])


_mesh = plsc.VectorSubcoreMesh(core_axis_name="c", subcore_axis_name="s")

_gine_call = functools.partial(
    pl.kernel,
    out_type=jax.ShapeDtypeStruct((N_NODES, 2, DH), jnp.float32),
    mesh=_mesh,
    scratch_types=[
        [pltpu.VMEM((B,), jnp.int32)] * 2,
        [pltpu.VMEM((B,), jnp.int32)] * 2,
        [pltpu.VMEM((B, DH), jnp.float32)] * 2,
        [pltpu.VMEM((B, DH), jnp.float32)] * 2,
        pltpu.VMEM_SHARED((N_NODES, DH), jnp.float32),
        pltpu.SemaphoreType.DMA,
        pltpu.SemaphoreType.DMA,
        pltpu.SemaphoreType.DMA,
        pltpu.SemaphoreType.DMA,
        pltpu.SemaphoreType.DMA,
        pltpu.SemaphoreType.DMA,
    ],
)(_gine_sc)


@jax.jit
def kernel(feat, edge_index, efeat):
    src = edge_index[0].astype(jnp.int32)
    dst = edge_index[1].astype(jnp.int32)
    feat_s = jnp.stack([feat[:, :DH], feat[:, DH:]])
    efeat3 = efeat.reshape(N_EDGES, 2, DH)
    out3 = _gine_call(feat_s, src, dst, efeat3)
    return out3.reshape(N_NODES, D_FEAT)
